# Initial kernel scaffold; baseline (speedup 1.0000x reference)
import jax
import jax.numpy as jnp
from jax.experimental import pallas as pl


def kernel(fea, edge_index, edge_weight, W1, b1, W2, b2):
    raise NotImplementedError("write your pallas kernel here")


# speedup vs baseline: 3.5541x; 3.5541x over previous
import functools

import jax
import jax.numpy as jnp
from jax import lax
from jax.experimental import pallas as pl
from jax.experimental.pallas import tpu as pltpu
from jax.experimental.pallas import tpu_sc as plsc

N_NODES = 10000
N_EDGES = 320000
D_IN, D_HID, D_OUT = 128, 128, 64

NC, NS, L = 2, 16, 16
NW = NC * NS
K = 128
C = -(-N_EDGES // (NW * K))
E_PAD = NW * K * C
RPS = 632
N_PAD = NS * RPS

ROW_BLK = 1000
GRID = N_NODES // ROW_BLK


def _mm_body(x_ref, w_ref, b_ref, o_ref):
    o_ref[...] = (
        jnp.dot(x_ref[...], w_ref[...], preferred_element_type=jnp.float32)
        + b_ref[...]
    )


def _mm(x, w, b):
    dout = w.shape[1]
    return pl.pallas_call(
        _mm_body,
        grid=(GRID,),
        in_specs=[
            pl.BlockSpec((ROW_BLK, x.shape[1]), lambda i: (i, 0)),
            pl.BlockSpec(w.shape, lambda i: (0, 0)),
            pl.BlockSpec((1, dout), lambda i: (0, 0)),
        ],
        out_specs=pl.BlockSpec((ROW_BLK, dout), lambda i: (i, 0)),
        out_shape=jax.ShapeDtypeStruct((x.shape[0], dout), jnp.float32),
    )(x, w, b.reshape(1, dout))


def _mm_fused_body(p_ref, w_ref, b_ref, o_ref):
    h = jnp.maximum(p_ref[0] + p_ref[1], 0.0)
    o_ref[...] = (
        jnp.dot(h, w_ref[...], preferred_element_type=jnp.float32) + b_ref[...]
    )


def _mm_fused(p, w, b):
    d = p.shape[2]
    dout = w.shape[1]
    return pl.pallas_call(
        _mm_fused_body,
        grid=(GRID,),
        in_specs=[
            pl.BlockSpec((2, ROW_BLK, d), lambda i: (0, i, 0)),
            pl.BlockSpec(w.shape, lambda i: (0, 0)),
            pl.BlockSpec((1, dout), lambda i: (0, 0)),
        ],
        out_specs=pl.BlockSpec((ROW_BLK, dout), lambda i: (i, 0)),
        out_shape=jax.ShapeDtypeStruct((N_NODES, dout), jnp.float32),
    )(p, w, b.reshape(1, dout))


def _pair_add_body(p_ref, o_ref):
    o_ref[...] = p_ref[0] + p_ref[1]


def _pair_add(p):
    d = p.shape[2]
    return pl.pallas_call(
        _pair_add_body,
        grid=(GRID,),
        in_specs=[pl.BlockSpec((2, ROW_BLK, d), lambda i: (0, i, 0))],
        out_specs=pl.BlockSpec((ROW_BLK, d), lambda i: (i, 0)),
        out_shape=jax.ShapeDtypeStruct((N_NODES, d), jnp.float32),
    )(p)


def _make_agg(D):
    mesh = plsc.VectorSubcoreMesh(core_axis_name="c", subcore_axis_name="s")

    @functools.partial(
        pl.kernel,
        out_type=jax.ShapeDtypeStruct((NC * N_PAD, D), jnp.float32),
        mesh=mesh,
        scratch_types=[
            pltpu.VMEM((K,), jnp.int32),
            pltpu.VMEM((K,), jnp.int32),
            pltpu.VMEM((K,), jnp.float32),
            pltpu.VMEM((K, D), jnp.float32),
            pltpu.VMEM_SHARED((N_PAD, D), jnp.float32),
            pltpu.SemaphoreType.DMA,
        ],
        compiler_params=pltpu.CompilerParams(use_tc_tiling_on_sc=False),
    )
    def agg(sup, src, dst, ew, zeros, out, src_v, dst_v, ew_v, rows_v, acc, sem):
        cid = lax.axis_index("c")
        sid = lax.axis_index("s")
        wid = sid * NC + cid
        row0 = sid * RPS
        pltpu.sync_copy(zeros.at[pl.ds(row0, RPS)], acc.at[pl.ds(row0, RPS)])
        plsc.subcore_barrier()
        base0 = wid * C * K

        def chunk(c, carry):
            base = base0 + c * K
            pltpu.sync_copy(src.at[pl.ds(base, K)], src_v)
            pltpu.sync_copy(dst.at[pl.ds(base, K)], dst_v)
            pltpu.sync_copy(ew.at[pl.ds(base, K)], ew_v)
            pltpu.async_copy(sup.at[src_v], rows_v, sem).wait()

            def scale(g, inner):
                ewg = ew_v[pl.ds(g * L, L)]
                for l in range(L):
                    w = ewg[l]
                    row = g * L + l
                    for j in range(D // L):
                        sl = pl.ds(j * L, L)
                        rows_v[row, sl] = rows_v[row, sl] * w
                return inner

            lax.fori_loop(0, K // L, scale, 0)
            pltpu.sync_copy(rows_v, acc.at[dst_v], add=True)
            return carry

        lax.fori_loop(0, C, chunk, 0)
        plsc.subcore_barrier()
        pltpu.sync_copy(
            acc.at[pl.ds(row0, RPS)],
            out.at[pl.ds(cid * N_PAD + row0, RPS)],
        )

    return agg


_agg_hid = _make_agg(D_HID)
_agg_out = _make_agg(D_OUT)


def kernel(fea, edge_index, edge_weight, W1, b1, W2, b2):
    pad = E_PAD - N_EDGES
    src = jnp.concatenate([edge_index[0], jnp.zeros((pad,), jnp.int32)])
    dst = jnp.concatenate([edge_index[1], jnp.zeros((pad,), jnp.int32)])
    ew = jnp.concatenate([edge_weight, jnp.zeros((pad,), jnp.float32)])
    z_hid = jnp.zeros((N_PAD, D_HID), jnp.float32)
    z_out = jnp.zeros((N_PAD, D_OUT), jnp.float32)

    sup1 = _mm(fea, W1, b1)
    p1 = _agg_hid(sup1, src, dst, ew, z_hid)
    sup2 = _mm_fused(p1.reshape(NC, N_PAD, D_HID), W2, b2)
    p2 = _agg_out(sup2, src, dst, ew, z_out)
    return _pair_add(p2.reshape(NC, N_PAD, D_OUT))


# speedup vs baseline: 3.7091x; 1.0436x over previous
import functools

import jax
import jax.numpy as jnp
from jax import lax
from jax.experimental import pallas as pl
from jax.experimental.pallas import tpu as pltpu
from jax.experimental.pallas import tpu_sc as plsc

N_NODES = 10000
N_EDGES = 320000
D_IN, D_HID, D_OUT = 128, 128, 64

NC, NS, L = 2, 16, 16
NW = NC * NS
K = 128
C = 80
E_PAD = NW * K * C
RPS = 632
N_PAD = NS * RPS

ROW_BLK = 1000
GRID = N_NODES // ROW_BLK


def _mm_body(x_ref, w_ref, b_ref, o_ref):
    o_ref[...] = (
        jnp.dot(x_ref[...], w_ref[...], preferred_element_type=jnp.float32)
        + b_ref[...]
    )


def _mm(x, w, b):
    dout = w.shape[1]
    return pl.pallas_call(
        _mm_body,
        grid=(GRID,),
        in_specs=[
            pl.BlockSpec((ROW_BLK, x.shape[1]), lambda i: (i, 0)),
            pl.BlockSpec(w.shape, lambda i: (0, 0)),
            pl.BlockSpec((1, dout), lambda i: (0, 0)),
        ],
        out_specs=pl.BlockSpec((ROW_BLK, dout), lambda i: (i, 0)),
        out_shape=jax.ShapeDtypeStruct((x.shape[0], dout), jnp.float32),
    )(x, w, b.reshape(1, dout))


def _mm_fused_body(p_ref, w_ref, b_ref, o_ref):
    h = jnp.maximum(p_ref[0] + p_ref[1], 0.0)
    o_ref[...] = (
        jnp.dot(h, w_ref[...], preferred_element_type=jnp.float32) + b_ref[...]
    )


def _mm_fused(p, w, b):
    d = p.shape[2]
    dout = w.shape[1]
    return pl.pallas_call(
        _mm_fused_body,
        grid=(GRID,),
        in_specs=[
            pl.BlockSpec((2, ROW_BLK, d), lambda i: (0, i, 0)),
            pl.BlockSpec(w.shape, lambda i: (0, 0)),
            pl.BlockSpec((1, dout), lambda i: (0, 0)),
        ],
        out_specs=pl.BlockSpec((ROW_BLK, dout), lambda i: (i, 0)),
        out_shape=jax.ShapeDtypeStruct((N_NODES, dout), jnp.float32),
    )(p, w, b.reshape(1, dout))


def _pair_add_body(p_ref, o_ref):
    o_ref[...] = p_ref[0] + p_ref[1]


def _pair_add(p):
    d = p.shape[2]
    return pl.pallas_call(
        _pair_add_body,
        grid=(GRID,),
        in_specs=[pl.BlockSpec((2, ROW_BLK, d), lambda i: (0, i, 0))],
        out_specs=pl.BlockSpec((ROW_BLK, d), lambda i: (i, 0)),
        out_shape=jax.ShapeDtypeStruct((N_NODES, d), jnp.float32),
    )(p)


def _make_agg(D):
    mesh = plsc.VectorSubcoreMesh(core_axis_name="c", subcore_axis_name="s")
    T = C // 4

    @functools.partial(
        pl.kernel,
        out_type=jax.ShapeDtypeStruct((NC * N_PAD, D), jnp.float32),
        mesh=mesh,
        scratch_types=[
            pltpu.VMEM((3, K), jnp.int32),
            pltpu.VMEM((3, K), jnp.int32),
            pltpu.VMEM((3, K), jnp.int32),
            pltpu.VMEM((3, K), jnp.int32),
            pltpu.VMEM((K, D), jnp.float32),
            pltpu.VMEM((K, D), jnp.float32),
            pltpu.VMEM_SHARED((N_PAD, D), jnp.float32),
            pltpu.SemaphoreType.DMA,
            pltpu.SemaphoreType.DMA,
            pltpu.SemaphoreType.DMA,
            pltpu.SemaphoreType.DMA,
            pltpu.SemaphoreType.DMA,
            pltpu.SemaphoreType.DMA,
            pltpu.SemaphoreType.DMA,
            pltpu.SemaphoreType.DMA,
        ],
        compiler_params=pltpu.CompilerParams(
            use_tc_tiling_on_sc=False, needs_layout_passes=False
        ),
    )
    def agg(sup, packed, zeros, out,
            i0, i1, i2, i3, rows0, rows1, acc,
            si0, si1, si2, si3, sg0, sg1, ss0, ss1):
        cid = lax.axis_index("c")
        sid = lax.axis_index("s")
        wid = sid * NC + cid
        row0 = sid * RPS
        pltpu.sync_copy(zeros.at[pl.ds(row0, RPS)], acc.at[pl.ds(row0, RPS)])
        plsc.subcore_barrier()
        g0 = wid * C

        idx_bufs = (i0, i1, i2, i3)
        idx_sems = (si0, si1, si2, si3)
        rows_bufs = (rows0, rows1)
        gather_sems = (sg0, sg1)
        scatter_sems = (ss0, ss1)

        def issue_idx(c, slot):
            pltpu.async_copy(packed.at[g0 + c], idx_bufs[slot], idx_sems[slot])

        def wait_idx(slot):
            pltpu.make_async_copy(
                packed.at[g0], idx_bufs[slot], idx_sems[slot]
            ).wait()

        def issue_gather(slot4, rslot):
            pltpu.async_copy(
                sup.at[idx_bufs[slot4].at[0]], rows_bufs[rslot],
                gather_sems[rslot],
            )

        def wait_gather(rslot):
            pltpu.make_async_copy(
                sup.at[pl.ds(0, K)], rows_bufs[rslot], gather_sems[rslot]
            ).wait()

        def issue_scatter(slot4, rslot):
            pltpu.async_copy(
                rows_bufs[rslot], acc.at[idx_bufs[slot4].at[1]],
                scatter_sems[rslot], add=True,
            )

        def wait_scatter(rslot):
            pltpu.make_async_copy(
                sup.at[pl.ds(0, K)], rows_bufs[rslot], scatter_sems[rslot]
            ).wait()

        def scale(rows_v, idx_v):
            def body(g, inner):
                ewg = plsc.bitcast(idx_v[2, pl.ds(g * L, L)], jnp.float32)
                for l in range(L):
                    w = ewg[l]
                    row = g * L + l
                    for j in range(D // L):
                        sl = pl.ds(j * L, L)
                        rows_v[row, sl] = rows_v[row, sl] * w
                return inner

            lax.fori_loop(0, K // L, body, 0)

        issue_idx(0, 0)
        issue_idx(1, 1)
        issue_idx(2, 2)
        wait_idx(0)
        issue_gather(0, 0)

        def step(t, carry):
            c = 4 * t
            not_last = t + 1 < T

            wait_gather(0)
            scale(rows0, i0)

            @pl.when(t > 0)
            def _():
                wait_scatter(1)
            issue_idx(c + 3, 3)
            wait_idx(1)
            issue_gather(1, 1)
            issue_scatter(0, 0)

            wait_gather(1)
            scale(rows1, i1)
            wait_scatter(0)
            @pl.when(not_last)
            def _():
                issue_idx(c + 4, 0)
            wait_idx(2)
            issue_gather(2, 0)
            issue_scatter(1, 1)

            wait_gather(0)
            scale(rows0, i2)
            wait_scatter(1)
            @pl.when(not_last)
            def _():
                issue_idx(c + 5, 1)
            wait_idx(3)
            issue_gather(3, 1)
            issue_scatter(2, 0)

            wait_gather(1)
            scale(rows1, i3)
            wait_scatter(0)
            @pl.when(not_last)
            def _():
                issue_idx(c + 6, 2)
                wait_idx(0)
                issue_gather(0, 0)
            issue_scatter(3, 1)

            return carry

        lax.fori_loop(0, T, step, 0)
        wait_scatter(1)

        plsc.subcore_barrier()
        pltpu.sync_copy(
            acc.at[pl.ds(row0, RPS)],
            out.at[pl.ds(cid * N_PAD + row0, RPS)],
        )

    return agg


_agg_hid = _make_agg(D_HID)
_agg_out = _make_agg(D_OUT)


def kernel(fea, edge_index, edge_weight, W1, b1, W2, b2):
    pad = E_PAD - N_EDGES
    src = jnp.concatenate([edge_index[0], jnp.zeros((pad,), jnp.int32)])
    dst = jnp.concatenate([edge_index[1], jnp.zeros((pad,), jnp.int32)])
    ewb = lax.bitcast_convert_type(
        jnp.concatenate([edge_weight, jnp.zeros((pad,), jnp.float32)]),
        jnp.int32,
    )
    packed = jnp.stack(
        [src.reshape(NW * C, K), dst.reshape(NW * C, K), ewb.reshape(NW * C, K)],
        axis=1,
    )
    z_hid = jnp.zeros((N_PAD, D_HID), jnp.float32)
    z_out = jnp.zeros((N_PAD, D_OUT), jnp.float32)

    sup1 = _mm(fea, W1, b1)
    p1 = _agg_hid(sup1, packed, z_hid)
    sup2 = _mm_fused(p1.reshape(NC, N_PAD, D_HID), W2, b2)
    p2 = _agg_out(sup2, packed, z_out)
    return _pair_add(p2.reshape(NC, N_PAD, D_OUT))


# speedup vs baseline: 3.8708x; 1.0436x over previous
import functools

import jax
import jax.numpy as jnp
from jax import lax
from jax.experimental import pallas as pl
from jax.experimental.pallas import tpu as pltpu
from jax.experimental.pallas import tpu_sc as plsc

N_NODES = 10000
N_EDGES = 320000
D_IN, D_HID, D_OUT = 128, 128, 64

NC, NS, L = 2, 16, 16
NW = NC * NS
K = 128
C = 80
E_PAD = NW * K * C
RPS = 632
N_PAD = NS * RPS

ROW_BLK = 1000
GRID = N_NODES // ROW_BLK


def _mm_body(x_ref, w_ref, b_ref, o_ref):
    o_ref[...] = (
        jnp.dot(x_ref[...], w_ref[...], preferred_element_type=jnp.float32)
        + b_ref[...]
    )


def _mm_split_body(x_ref, w_ref, b_ref, o_ref):
    o_ref[0] = (
        jnp.dot(x_ref[...], w_ref[0], preferred_element_type=jnp.float32)
        + b_ref[0]
    )


def _mm_split(x, w, b):
    dout = w.shape[1]
    dh = dout // 2
    din = x.shape[1]
    ws = jnp.stack([w[:, :dh], w[:, dh:]])
    bs = b.reshape(2, 1, dh)
    return pl.pallas_call(
        _mm_split_body,
        grid=(GRID, 2),
        in_specs=[
            pl.BlockSpec((ROW_BLK, din), lambda i, j: (i, 0)),
            pl.BlockSpec((1, din, dh), lambda i, j: (j, 0, 0)),
            pl.BlockSpec((1, 1, dh), lambda i, j: (j, 0, 0)),
        ],
        out_specs=pl.BlockSpec((1, ROW_BLK, dh), lambda i, j: (j, i, 0)),
        out_shape=jax.ShapeDtypeStruct((2, x.shape[0], dh), jnp.float32),
    )(x, ws, bs)


def _mm_fused_body(p_ref, w_ref, b_ref, o_ref):
    dh = p_ref.shape[2]
    h_lo = jnp.maximum(p_ref[0], 0.0)
    h_hi = jnp.maximum(p_ref[1], 0.0)
    o_ref[...] = (
        jnp.dot(h_lo, w_ref[:dh], preferred_element_type=jnp.float32)
        + jnp.dot(h_hi, w_ref[dh:], preferred_element_type=jnp.float32)
        + b_ref[...]
    )


def _mm_fused(p, w, b):
    d = p.shape[2]
    dout = w.shape[1]
    return pl.pallas_call(
        _mm_fused_body,
        grid=(GRID,),
        in_specs=[
            pl.BlockSpec((2, ROW_BLK, d), lambda i: (0, i, 0)),
            pl.BlockSpec(w.shape, lambda i: (0, 0)),
            pl.BlockSpec((1, dout), lambda i: (0, 0)),
        ],
        out_specs=pl.BlockSpec((ROW_BLK, dout), lambda i: (i, 0)),
        out_shape=jax.ShapeDtypeStruct((N_NODES, dout), jnp.float32),
    )(p, w, b.reshape(1, dout))


def _pair_add_body(p_ref, o_ref):
    o_ref[...] = p_ref[0] + p_ref[1]


def _pair_add(p):
    d = p.shape[2]
    return pl.pallas_call(
        _pair_add_body,
        grid=(GRID,),
        in_specs=[pl.BlockSpec((2, ROW_BLK, d), lambda i: (0, i, 0))],
        out_specs=pl.BlockSpec((ROW_BLK, d), lambda i: (i, 0)),
        out_shape=jax.ShapeDtypeStruct((N_NODES, d), jnp.float32),
    )(p)


def _make_agg(D, dsplit):
    mesh = plsc.VectorSubcoreMesh(core_axis_name="c", subcore_axis_name="s")
    TC_CHUNKS = (NW * C) // NS if dsplit else C
    T = TC_CHUNKS // 4
    SUP_ROWS = NC * N_NODES if dsplit else N_NODES

    @functools.partial(
        pl.kernel,
        out_type=jax.ShapeDtypeStruct((NC * N_PAD, D), jnp.float32),
        mesh=mesh,
        scratch_types=[
            pltpu.VMEM((3, K), jnp.int32),
            pltpu.VMEM((3, K), jnp.int32),
            pltpu.VMEM((3, K), jnp.int32),
            pltpu.VMEM((3, K), jnp.int32),
            pltpu.VMEM((K, D), jnp.float32),
            pltpu.VMEM((K, D), jnp.float32),
            pltpu.VMEM((K, D), jnp.float32),
            pltpu.VMEM((K, D), jnp.float32),
            pltpu.VMEM_SHARED((N_PAD, D), jnp.float32),
            pltpu.SemaphoreType.DMA,
            pltpu.SemaphoreType.DMA,
            pltpu.SemaphoreType.DMA,
            pltpu.SemaphoreType.DMA,
            pltpu.SemaphoreType.DMA,
            pltpu.SemaphoreType.DMA,
            pltpu.SemaphoreType.DMA,
            pltpu.SemaphoreType.DMA,
        ],
        compiler_params=pltpu.CompilerParams(
            use_tc_tiling_on_sc=False, needs_layout_passes=False
        ),
    )
    def agg(sup, packed, zeros, out,
            i0, i1, i2, i3, rows0, rows1, scaled0, scaled1, acc,
            si0, si1, si2, si3, sg0, sg1, ss0, ss1):
        cid = lax.axis_index("c")
        sid = lax.axis_index("s")
        wid = sid * NC + cid
        row0 = sid * RPS
        pltpu.sync_copy(zeros.at[pl.ds(row0, RPS)], acc.at[pl.ds(row0, RPS)])
        plsc.subcore_barrier()
        g0 = sid * TC_CHUNKS if dsplit else wid * C
        src_off = cid * N_NODES

        idx_bufs = (i0, i1, i2, i3)
        idx_sems = (si0, si1, si2, si3)
        rows_bufs = (rows0, rows1)
        scaled_bufs = (scaled0, scaled1)
        gather_sems = (sg0, sg1)
        scatter_sems = (ss0, ss1)

        def issue_idx(c, slot):
            pltpu.async_copy(packed.at[g0 + c], idx_bufs[slot], idx_sems[slot])

        def wait_idx(slot):
            pltpu.make_async_copy(
                packed.at[g0], idx_bufs[slot], idx_sems[slot]
            ).wait()
            if dsplit:
                for q in range(K // L):
                    sl = pl.ds(q * L, L)
                    idx_bufs[slot][0, sl] = idx_bufs[slot][0, sl] + src_off

        def issue_gather(slot4, rslot):
            pltpu.async_copy(
                sup.at[idx_bufs[slot4].at[0]], rows_bufs[rslot],
                gather_sems[rslot],
            )

        def wait_gather(rslot):
            pltpu.make_async_copy(
                sup.at[pl.ds(0, K)], rows_bufs[rslot], gather_sems[rslot]
            ).wait()

        def issue_scatter(slot4, rslot):
            pltpu.async_copy(
                scaled_bufs[rslot], acc.at[idx_bufs[slot4].at[1]],
                scatter_sems[rslot], add=True,
            )

        def wait_scatter(rslot):
            pltpu.make_async_copy(
                sup.at[pl.ds(0, K)], scaled_bufs[rslot], scatter_sems[rslot]
            ).wait()

        def scale(rows_v, out_v, idx_v):
            @plsc.parallel_loop(0, K // L)
            def body(g):
                ewg = plsc.bitcast(idx_v[2, pl.ds(g * L, L)], jnp.float32)
                for l in range(L):
                    w = ewg[l]
                    row = g * L + l
                    for j in range(D // L):
                        sl = pl.ds(j * L, L)
                        out_v[row, sl] = rows_v[row, sl] * w

        issue_idx(0, 0)
        issue_idx(1, 1)
        issue_idx(2, 2)
        wait_idx(0)
        issue_gather(0, 0)

        def step(t, carry):
            c = 4 * t
            not_last = t + 1 < T

            wait_gather(0)
            scale(rows0, scaled0, i0)

            @pl.when(t > 0)
            def _():
                wait_scatter(1)
            issue_idx(c + 3, 3)
            wait_idx(1)
            issue_gather(1, 1)
            issue_scatter(0, 0)

            wait_gather(1)
            scale(rows1, scaled1, i1)
            wait_scatter(0)
            @pl.when(not_last)
            def _():
                issue_idx(c + 4, 0)
            wait_idx(2)
            issue_gather(2, 0)
            issue_scatter(1, 1)

            wait_gather(0)
            scale(rows0, scaled0, i2)
            wait_scatter(1)
            @pl.when(not_last)
            def _():
                issue_idx(c + 5, 1)
            wait_idx(3)
            issue_gather(3, 1)
            issue_scatter(2, 0)

            wait_gather(1)
            scale(rows1, scaled1, i3)
            wait_scatter(0)
            @pl.when(not_last)
            def _():
                issue_idx(c + 6, 2)
                wait_idx(0)
                issue_gather(0, 0)
            issue_scatter(3, 1)

            return carry

        lax.fori_loop(0, T, step, 0)
        wait_scatter(1)

        plsc.subcore_barrier()
        pltpu.sync_copy(
            acc.at[pl.ds(row0, RPS)],
            out.at[pl.ds(cid * N_PAD + row0, RPS)],
        )

    return agg


_agg_hid = _make_agg(D_HID // 2, dsplit=True)
_agg_out = _make_agg(D_OUT, dsplit=False)


def kernel(fea, edge_index, edge_weight, W1, b1, W2, b2):
    pad = E_PAD - N_EDGES
    src = jnp.concatenate([edge_index[0], jnp.zeros((pad,), jnp.int32)])
    dst = jnp.concatenate([edge_index[1], jnp.zeros((pad,), jnp.int32)])
    ewb = lax.bitcast_convert_type(
        jnp.concatenate([edge_weight, jnp.zeros((pad,), jnp.float32)]),
        jnp.int32,
    )
    packed = jnp.stack(
        [src.reshape(NW * C, K), dst.reshape(NW * C, K), ewb.reshape(NW * C, K)],
        axis=1,
    )
    z64 = jnp.zeros((N_PAD, 64), jnp.float32)

    sup1 = _mm_split(fea, W1, b1).reshape(NC * N_NODES, D_HID // 2)
    h_halves = _agg_hid(sup1, packed, z64)
    sup2 = _mm_fused(h_halves.reshape(NC, N_PAD, D_HID // 2), W2, b2)
    p2 = _agg_out(sup2, packed, z64)
    return _pair_add(p2.reshape(NC, N_PAD, D_OUT))


# speedup vs baseline: 4.6102x; 1.1910x over previous
import functools

import jax
import jax.numpy as jnp
from jax import lax
from jax.experimental import pallas as pl
from jax.experimental.pallas import tpu as pltpu
from jax.experimental.pallas import tpu_sc as plsc

N_NODES = 10000
N_EDGES = 320000
D_IN, D_HID, D_OUT = 128, 128, 64

NC, NS, L = 2, 16, 16
NW = NC * NS
K = 128
C = 80
E_PAD = NW * K * C
RPS = 632
N_PAD = NS * RPS

ROW_BLK = 1000
GRID = N_NODES // ROW_BLK


def _mm_body(x_ref, w_ref, b_ref, o_ref):
    o_ref[...] = (
        jnp.dot(x_ref[...], w_ref[...], preferred_element_type=jnp.float32)
        + b_ref[...]
    )


def _mm_split_body(x_ref, w_ref, b_ref, o_ref):
    o_ref[0] = (
        jnp.dot(x_ref[...], w_ref[0], preferred_element_type=jnp.float32)
        + b_ref[0]
    )


def _mm_split(x, w, b):
    dout = w.shape[1]
    dh = dout // 2
    din = x.shape[1]
    ws = jnp.stack([w[:, :dh], w[:, dh:]])
    bs = b.reshape(2, 1, dh)
    return pl.pallas_call(
        _mm_split_body,
        grid=(GRID, 2),
        in_specs=[
            pl.BlockSpec((ROW_BLK, din), lambda i, j: (i, 0)),
            pl.BlockSpec((1, din, dh), lambda i, j: (j, 0, 0)),
            pl.BlockSpec((1, 1, dh), lambda i, j: (j, 0, 0)),
        ],
        out_specs=pl.BlockSpec((1, ROW_BLK, dh), lambda i, j: (j, i, 0)),
        out_shape=jax.ShapeDtypeStruct((2, x.shape[0], dh), jnp.float32),
    )(x, ws, bs)


def _mm_fused_body(p_ref, w_ref, b_ref, o_ref):
    dh = p_ref.shape[2]
    h_lo = jnp.maximum(p_ref[0], 0.0)
    h_hi = jnp.maximum(p_ref[1], 0.0)
    o_ref[...] = (
        jnp.dot(h_lo, w_ref[:dh], preferred_element_type=jnp.float32)
        + jnp.dot(h_hi, w_ref[dh:], preferred_element_type=jnp.float32)
        + b_ref[...]
    )


def _mm_fused(p, w, b):
    d = p.shape[2]
    dout = w.shape[1]
    return pl.pallas_call(
        _mm_fused_body,
        grid=(GRID,),
        in_specs=[
            pl.BlockSpec((2, ROW_BLK, d), lambda i: (0, i, 0)),
            pl.BlockSpec(w.shape, lambda i: (0, 0)),
            pl.BlockSpec((1, dout), lambda i: (0, 0)),
        ],
        out_specs=pl.BlockSpec((ROW_BLK, dout), lambda i: (i, 0)),
        out_shape=jax.ShapeDtypeStruct((N_NODES, dout), jnp.float32),
    )(p, w, b.reshape(1, dout))


def _pair_add_body(p_ref, o_ref):
    o_ref[...] = p_ref[0] + p_ref[1]


def _pair_add(p):
    d = p.shape[2]
    return pl.pallas_call(
        _pair_add_body,
        grid=(GRID,),
        in_specs=[pl.BlockSpec((2, ROW_BLK, d), lambda i: (0, i, 0))],
        out_specs=pl.BlockSpec((ROW_BLK, d), lambda i: (i, 0)),
        out_shape=jax.ShapeDtypeStruct((N_NODES, d), jnp.float32),
    )(p)


def _make_agg(D, dsplit):
    mesh = plsc.VectorSubcoreMesh(core_axis_name="c", subcore_axis_name="s")
    TC_CHUNKS = (NW * C) // NS if dsplit else C
    T = TC_CHUNKS // 4
    SUP_ROWS = NC * N_NODES if dsplit else N_NODES

    @functools.partial(
        pl.kernel,
        out_type=jax.ShapeDtypeStruct((NC * N_PAD, D), jnp.float32),
        mesh=mesh,
        scratch_types=[
            pltpu.VMEM((3, K), jnp.int32),
            pltpu.VMEM((3, K), jnp.int32),
            pltpu.VMEM((3, K), jnp.int32),
            pltpu.VMEM((3, K), jnp.int32),
            pltpu.VMEM((K, D), jnp.float32),
            pltpu.VMEM((K, D), jnp.float32),
            pltpu.VMEM((K, D), jnp.float32),
            pltpu.VMEM((K, D), jnp.float32),
            pltpu.VMEM_SHARED((N_PAD, D), jnp.float32),
            pltpu.SemaphoreType.DMA,
            pltpu.SemaphoreType.DMA,
            pltpu.SemaphoreType.DMA,
            pltpu.SemaphoreType.DMA,
            pltpu.SemaphoreType.DMA,
            pltpu.SemaphoreType.DMA,
            pltpu.SemaphoreType.DMA,
            pltpu.SemaphoreType.DMA,
        ],
        compiler_params=pltpu.CompilerParams(
            use_tc_tiling_on_sc=False, needs_layout_passes=False
        ),
    )
    def agg(sup, packed, zeros, out,
            i0, i1, i2, i3, rows0, rows1, scaled0, scaled1, acc,
            si0, si1, si2, si3, sg0, sg1, ss0, ss1):
        cid = lax.axis_index("c")
        sid = lax.axis_index("s")
        wid = sid * NC + cid
        row0 = sid * RPS
        pltpu.sync_copy(zeros.at[pl.ds(row0, RPS)], acc.at[pl.ds(row0, RPS)])
        plsc.subcore_barrier()
        g0 = sid * TC_CHUNKS if dsplit else wid * C
        src_off = cid * N_NODES

        idx_bufs = (i0, i1, i2, i3)
        idx_sems = (si0, si1, si2, si3)
        rows_bufs = (rows0, rows1)
        scaled_bufs = (scaled0, scaled1)
        gather_sems = (sg0, sg1)
        scatter_sems = (ss0, ss1)

        def issue_idx(c, slot):
            pltpu.async_copy(packed.at[g0 + c], idx_bufs[slot], idx_sems[slot])

        def wait_idx(slot):
            pltpu.make_async_copy(
                packed.at[g0], idx_bufs[slot], idx_sems[slot]
            ).wait()
            if dsplit:
                for q in range(K // L):
                    sl = pl.ds(q * L, L)
                    idx_bufs[slot][0, sl] = idx_bufs[slot][0, sl] + src_off

        def issue_gather(slot4, rslot):
            pltpu.async_copy(
                sup.at[idx_bufs[slot4].at[0]], rows_bufs[rslot],
                gather_sems[rslot],
            )

        def wait_gather(rslot):
            pltpu.make_async_copy(
                sup.at[pl.ds(0, K)], rows_bufs[rslot], gather_sems[rslot]
            ).wait()

        def issue_scatter(slot4, rslot):
            pltpu.async_copy(
                scaled_bufs[rslot], acc.at[idx_bufs[slot4].at[1]],
                scatter_sems[rslot], add=True,
            )

        def wait_scatter(rslot):
            pltpu.make_async_copy(
                sup.at[pl.ds(0, K)], scaled_bufs[rslot], scatter_sems[rslot]
            ).wait()

        def scale(rows_v, out_v, idx_v):
            @plsc.parallel_loop(0, K // L)
            def body(g):
                ewg = plsc.bitcast(idx_v[2, pl.ds(g * L, L)], jnp.float32)
                for l in range(L):
                    w = ewg[l]
                    row = g * L + l
                    for j in range(D // L):
                        sl = pl.ds(j * L, L)
                        out_v[row, sl] = rows_v[row, sl] * w

        issue_idx(0, 0)
        issue_idx(1, 1)
        wait_idx(0)
        issue_gather(0, 0)

        def block(c, q):
            X = q % 2

            @pl.when(c + 1 < TC_CHUNKS)
            def _():
                wait_idx((q + 1) % 4)
                issue_gather((q + 1) % 4, (q + 1) % 2)

            @pl.when(c >= 2)
            def _():
                wait_scatter(X)

            @pl.when(c + 2 < TC_CHUNKS)
            def _():
                issue_idx(c + 2, (q + 2) % 4)

            wait_gather(X)
            scale(rows_bufs[X], scaled_bufs[X], idx_bufs[q])
            issue_scatter(q, X)

        def step(t, carry):
            c = 4 * t
            for q in range(4):
                block(c + q, q)
            return carry

        lax.fori_loop(0, T, step, 0)
        wait_scatter(0)
        wait_scatter(1)

        plsc.subcore_barrier()
        pltpu.sync_copy(
            acc.at[pl.ds(row0, RPS)],
            out.at[pl.ds(cid * N_PAD + row0, RPS)],
        )

    return agg


_agg_hid = _make_agg(D_HID // 2, dsplit=True)
_agg_out = _make_agg(D_OUT, dsplit=False)


def kernel(fea, edge_index, edge_weight, W1, b1, W2, b2):
    pad = E_PAD - N_EDGES
    src = jnp.concatenate([edge_index[0], jnp.zeros((pad,), jnp.int32)])
    dst = jnp.concatenate([edge_index[1], jnp.zeros((pad,), jnp.int32)])
    ewb = lax.bitcast_convert_type(
        jnp.concatenate([edge_weight, jnp.zeros((pad,), jnp.float32)]),
        jnp.int32,
    )
    packed = jnp.stack(
        [src.reshape(NW * C, K), dst.reshape(NW * C, K), ewb.reshape(NW * C, K)],
        axis=1,
    )
    z64 = jnp.zeros((N_PAD, 64), jnp.float32)

    sup1 = _mm_split(fea, W1, b1).reshape(NC * N_NODES, D_HID // 2)
    h_halves = _agg_hid(sup1, packed, z64)
    sup2 = _mm_fused(h_halves.reshape(NC, N_PAD, D_HID // 2), W2, b2)
    p2 = _agg_out(sup2, packed, z64)
    return _pair_add(p2.reshape(NC, N_PAD, D_OUT))


# speedup vs baseline: 9.3293x; 2.0236x over previous
import functools

import jax
import jax.numpy as jnp
from jax import lax
from jax.experimental import pallas as pl
from jax.experimental.pallas import tpu as pltpu
from jax.experimental.pallas import tpu_sc as plsc

N_NODES = 10000
N_EDGES = 320000
D_IN, D_HID, D_OUT = 128, 128, 64

NC, NS, L = 2, 16, 16
NW = NC * NS
K = 128
C = 80
E_PAD = NW * K * C
RPS = 632
N_PAD = NS * RPS

ROW_BLK = 1000
GRID = N_NODES // ROW_BLK


def _mm_body(x_ref, w_ref, b_ref, o_ref):
    o_ref[...] = (
        jnp.dot(x_ref[...], w_ref[...], preferred_element_type=jnp.float32)
        + b_ref[...]
    )


def _mm_split_body(x_ref, w_ref, b_ref, o_ref):
    o_ref[0] = (
        jnp.dot(x_ref[...], w_ref[0], preferred_element_type=jnp.float32)
        + b_ref[0]
    )


def _mm_split(x, w, b):
    dout = w.shape[1]
    dh = dout // 2
    din = x.shape[1]
    ws = jnp.stack([w[:, :dh], w[:, dh:]])
    bs = b.reshape(2, 1, dh)
    return pl.pallas_call(
        _mm_split_body,
        grid=(GRID, 2),
        in_specs=[
            pl.BlockSpec((ROW_BLK, din), lambda i, j: (i, 0)),
            pl.BlockSpec((1, din, dh), lambda i, j: (j, 0, 0)),
            pl.BlockSpec((1, 1, dh), lambda i, j: (j, 0, 0)),
        ],
        out_specs=pl.BlockSpec((1, ROW_BLK, dh), lambda i, j: (j, i, 0)),
        out_shape=jax.ShapeDtypeStruct((2, N_PAD, dh), jnp.float32),
    )(x, ws, bs)


def _mm_fused_body(p_ref, w_ref, b_ref, o_ref):
    dh = p_ref.shape[2]
    h_lo = jnp.maximum(p_ref[0], 0.0)
    h_hi = jnp.maximum(p_ref[1], 0.0)
    o_ref[...] = (
        jnp.dot(h_lo, w_ref[:dh], preferred_element_type=jnp.float32)
        + jnp.dot(h_hi, w_ref[dh:], preferred_element_type=jnp.float32)
        + b_ref[...]
    )


def _mm_fused(p, w, b):
    d = p.shape[2]
    dout = w.shape[1]
    return pl.pallas_call(
        _mm_fused_body,
        grid=(GRID,),
        in_specs=[
            pl.BlockSpec((2, ROW_BLK, d), lambda i: (0, i, 0)),
            pl.BlockSpec(w.shape, lambda i: (0, 0)),
            pl.BlockSpec((1, dout), lambda i: (0, 0)),
        ],
        out_specs=pl.BlockSpec((ROW_BLK, dout), lambda i: (i, 0)),
        out_shape=jax.ShapeDtypeStruct((N_PAD, dout), jnp.float32),
    )(p, w, b.reshape(1, dout))


def _pair_add_body(p_ref, o_ref):
    o_ref[...] = p_ref[0] + p_ref[1]


def _pair_add(p):
    d = p.shape[2]
    return pl.pallas_call(
        _pair_add_body,
        grid=(GRID,),
        in_specs=[pl.BlockSpec((2, ROW_BLK, d), lambda i: (0, i, 0))],
        out_specs=pl.BlockSpec((ROW_BLK, d), lambda i: (i, 0)),
        out_shape=jax.ShapeDtypeStruct((N_NODES, d), jnp.float32),
    )(p)


def _make_agg(D, dsplit):
    mesh = plsc.VectorSubcoreMesh(core_axis_name="c", subcore_axis_name="s")
    TC_CHUNKS = (NW * C) // NS if dsplit else C
    T = TC_CHUNKS // 4

    @functools.partial(
        pl.kernel,
        out_type=jax.ShapeDtypeStruct((NC * N_PAD, D), jnp.float32),
        mesh=mesh,
        scratch_types=[
            pltpu.VMEM((3, K), jnp.int32),
            pltpu.VMEM((3, K), jnp.int32),
            pltpu.VMEM((3, K), jnp.int32),
            pltpu.VMEM((3, K), jnp.int32),
            pltpu.VMEM((K, D), jnp.float32),
            pltpu.VMEM((K, D), jnp.float32),
            pltpu.VMEM((K, D), jnp.float32),
            pltpu.VMEM((K, D), jnp.float32),
            pltpu.VMEM_SHARED((N_PAD, D), jnp.float32),
            pltpu.VMEM_SHARED((N_PAD, D), jnp.float32),
            pltpu.SemaphoreType.DMA,
            pltpu.SemaphoreType.DMA,
            pltpu.SemaphoreType.DMA,
            pltpu.SemaphoreType.DMA,
            pltpu.SemaphoreType.DMA,
            pltpu.SemaphoreType.DMA,
            pltpu.SemaphoreType.DMA,
            pltpu.SemaphoreType.DMA,
        ],
        compiler_params=pltpu.CompilerParams(
            use_tc_tiling_on_sc=False, needs_layout_passes=False
        ),
    )
    def agg(sup, packed, zeros, out,
            i0, i1, i2, i3, rows0, rows1, scaled0, scaled1, acc, sup_sp,
            si0, si1, si2, si3, sg0, sg1, ss0, ss1):
        cid = lax.axis_index("c")
        sid = lax.axis_index("s")
        wid = sid * NC + cid
        row0 = sid * RPS
        pltpu.sync_copy(zeros.at[pl.ds(row0, RPS)], acc.at[pl.ds(row0, RPS)])
        sup_base = cid * N_PAD + row0 if dsplit else row0
        pltpu.sync_copy(sup.at[pl.ds(sup_base, RPS)], sup_sp.at[pl.ds(row0, RPS)])
        plsc.subcore_barrier()
        g0 = sid * TC_CHUNKS if dsplit else wid * C

        idx_bufs = (i0, i1, i2, i3)
        idx_sems = (si0, si1, si2, si3)
        rows_bufs = (rows0, rows1)
        scaled_bufs = (scaled0, scaled1)
        gather_sems = (sg0, sg1)
        scatter_sems = (ss0, ss1)

        def issue_idx(c, slot):
            pltpu.async_copy(packed.at[g0 + c], idx_bufs[slot], idx_sems[slot])

        def wait_idx(slot):
            pltpu.make_async_copy(
                packed.at[g0], idx_bufs[slot], idx_sems[slot]
            ).wait()

        def issue_gather(slot4, rslot):
            pltpu.async_copy(
                sup_sp.at[idx_bufs[slot4].at[0]], rows_bufs[rslot],
                gather_sems[rslot],
            )

        def wait_gather(rslot):
            pltpu.make_async_copy(
                sup.at[pl.ds(0, K)], rows_bufs[rslot], gather_sems[rslot]
            ).wait()

        def issue_scatter(slot4, rslot):
            pltpu.async_copy(
                scaled_bufs[rslot], acc.at[idx_bufs[slot4].at[1]],
                scatter_sems[rslot], add=True,
            )

        def wait_scatter(rslot):
            pltpu.make_async_copy(
                sup.at[pl.ds(0, K)], scaled_bufs[rslot], scatter_sems[rslot]
            ).wait()

        def scale(rows_v, out_v, idx_v):
            @plsc.parallel_loop(0, K // L)
            def body(g):
                ewg = plsc.bitcast(idx_v[2, pl.ds(g * L, L)], jnp.float32)
                for l in range(L):
                    w = ewg[l]
                    row = g * L + l
                    for j in range(D // L):
                        sl = pl.ds(j * L, L)
                        out_v[row, sl] = rows_v[row, sl] * w

        issue_idx(0, 0)
        issue_idx(1, 1)
        wait_idx(0)
        issue_gather(0, 0)

        def block(c, q):
            X = q % 2

            @pl.when(c + 1 < TC_CHUNKS)
            def _():
                wait_idx((q + 1) % 4)
                issue_gather((q + 1) % 4, (q + 1) % 2)

            @pl.when(c >= 2)
            def _():
                wait_scatter(X)

            @pl.when(c + 2 < TC_CHUNKS)
            def _():
                issue_idx(c + 2, (q + 2) % 4)

            wait_gather(X)
            scale(rows_bufs[X], scaled_bufs[X], idx_bufs[q])
            issue_scatter(q, X)

        def step(t, carry):
            c = 4 * t
            for q in range(4):
                block(c + q, q)
            return carry

        lax.fori_loop(0, T, step, 0)
        wait_scatter(0)
        wait_scatter(1)

        plsc.subcore_barrier()
        pltpu.sync_copy(
            acc.at[pl.ds(row0, RPS)],
            out.at[pl.ds(cid * N_PAD + row0, RPS)],
        )

    return agg


_agg_hid = _make_agg(D_HID // 2, dsplit=True)
_agg_out = _make_agg(D_OUT, dsplit=False)


def kernel(fea, edge_index, edge_weight, W1, b1, W2, b2):
    pad = E_PAD - N_EDGES
    src = jnp.concatenate([edge_index[0], jnp.zeros((pad,), jnp.int32)])
    dst = jnp.concatenate([edge_index[1], jnp.zeros((pad,), jnp.int32)])
    ewb = lax.bitcast_convert_type(
        jnp.concatenate([edge_weight, jnp.zeros((pad,), jnp.float32)]),
        jnp.int32,
    )
    packed = jnp.stack(
        [src.reshape(NW * C, K), dst.reshape(NW * C, K), ewb.reshape(NW * C, K)],
        axis=1,
    )
    z64 = jnp.zeros((N_PAD, 64), jnp.float32)

    sup1 = _mm_split(fea, W1, b1).reshape(NC * N_PAD, D_HID // 2)
    h_halves = _agg_hid(sup1, packed, z64)
    sup2 = _mm_fused(h_halves.reshape(NC, N_PAD, D_HID // 2), W2, b2)
    p2 = _agg_out(sup2, packed, z64)
    return _pair_add(p2.reshape(NC, N_PAD, D_OUT))
